# baseline (speedup 1.0000x reference)
import jax
import jax.numpy as jnp
from jax import lax
from jax.experimental import pallas as pl
from jax.experimental.pallas import tpu as pltpu
from jax.experimental.pallas import tpu_sc as plsc

N = 6400000
N_EVENTS = 1024
N_STATIONS = 40
N_ORI = 2
NG = N_EVENTS * N_STATIONS * N_ORI

NC = 2
NS = 16
NW = NC * NS

BLK = 2048
NBLK = N // BLK
BASE_BLKS = NBLK // NW
EXTRA = NBLK - BASE_BLKS * NW
KF = BLK // 128

_mesh = plsc.VectorSubcoreMesh(core_axis_name="c", subcore_axis_name="s")


def _accum_body(xy_hbm, z_hbm, o_hbm, b_hbm, zero_hbm, out_hbm, g_hbm,
                cnt_t, sum_t, sq_t, xyb0, xyb1, zb0, zb1, ob0, ob1, bb0, bb1,
                sqb0, sqb1, onesb, idx128, insem, ssem, gsem):
    cid = lax.axis_index("c")
    sid = lax.axis_index("s")
    wid = sid * NC + cid
    nblk = BASE_BLKS + jnp.where(wid < EXTRA, 1, 0)
    xyb = (xyb0, xyb1)
    zb = (zb0, zb1)
    ob = (ob0, ob1)
    bb = (bb0, bb1)
    sqb = (sqb0, sqb1)
    ones = jnp.full((16,), 1.0, jnp.float32)

    zrows = NG // NS
    for t in (cnt_t, sum_t, sq_t):
        pltpu.sync_copy(zero_hbm, t.at[pl.ds(sid * zrows, zrows)])

    def _prefill(j, _):
        onesb[pl.ds(j * 16, 16)] = ones
        return 0

    lax.fori_loop(0, BLK // 16, _prefill, 0)

    plsc.subcore_barrier()

    def _hb(i):
        return (wid + NW * i) * BLK

    def _fire_in(i, slot):
        sl = pl.ds(_hb(i), BLK)
        pltpu.async_copy(xy_hbm.at[sl], xyb[slot], insem)
        pltpu.async_copy(z_hbm.at[sl], zb[slot], insem)
        pltpu.async_copy(o_hbm.at[sl], ob[slot], insem)
        pltpu.async_copy(b_hbm.at[sl], bb[slot], insem)

    def _wait_in(slot):
        sl = pl.ds(0, BLK)
        pltpu.make_async_copy(xy_hbm.at[sl], xyb[slot], insem).wait()
        pltpu.make_async_copy(z_hbm.at[sl], zb[slot], insem).wait()
        pltpu.make_async_copy(o_hbm.at[sl], ob[slot], insem).wait()
        pltpu.make_async_copy(b_hbm.at[sl], bb[slot], insem).wait()

    def _step(i, slot, fire_next):
        _wait_in(slot)

        @pl.when(fire_next)
        def _():
            _fire_in(i + 1, 1 - slot)

        def _vreg(j, _):
            sl16 = pl.ds(j * 16, 16)
            b = bb[slot][sl16]
            z = zb[slot][sl16]
            o = ob[slot][sl16]
            xv = xyb[slot][sl16]
            g = (b * N_STATIONS + z) * N_ORI + o
            idx128[slot, j // 8, pl.ds((j % 8) * 16, 16)] = g
            sqb[slot][sl16] = xv * xv
            return 0

        descs = []
        for m in range(4):
            lax.fori_loop(m * (BLK // 64), (m + 1) * (BLK // 64), _vreg, 0)
            for k in range(m * (KF // 4), (m + 1) * (KF // 4)):
                sl = pl.ds(k * 128, 128)
                irow = idx128.at[slot, k]
                descs.append(pltpu.async_copy(
                    onesb.at[sl], cnt_t.at[irow], ssem, add=True))
                descs.append(pltpu.async_copy(
                    xyb[slot].at[sl], sum_t.at[irow], ssem, add=True))
                descs.append(pltpu.async_copy(
                    sqb[slot].at[sl], sq_t.at[irow], ssem, add=True))

        gdst = g_hbm.at[pl.ds((wid + NW * i) * KF, KF), :]
        pltpu.async_copy(idx128.at[slot, pl.ds(0, KF)], gdst, gsem)

        for d in descs:
            d.wait()
        pltpu.make_async_copy(
            idx128.at[slot, pl.ds(0, KF)], gdst, gsem).wait()

    _fire_in(0, 0)

    def _pair(i2, _):
        i = i2 * 2
        _step(i, 0, i + 1 < nblk)
        _step(i + 1, 1, i + 2 < nblk)
        return 0

    lax.fori_loop(0, nblk // 2, _pair, 0)

    @pl.when(lax.rem(nblk, 2) == 1)
    def _():
        _step(nblk - 1, 0, jnp.bool_(False))

    plsc.subcore_barrier()

    for comp, t in enumerate((cnt_t, sum_t, sq_t)):
        pltpu.sync_copy(
            t.at[pl.ds(sid * zrows, zrows)],
            out_hbm.at[pl.ds((cid * 3 + comp) * NG + sid * zrows, zrows)])


_accum = pl.kernel(
    _accum_body,
    out_type=(jax.ShapeDtypeStruct((NC * 3 * NG,), jnp.float32),
              jax.ShapeDtypeStruct((N // 128, 128), jnp.int32)),
    mesh=_mesh,
    compiler_params=pltpu.CompilerParams(needs_layout_passes=False),
    scratch_types=[
        pltpu.VMEM_SHARED((NG,), jnp.float32),
        pltpu.VMEM_SHARED((NG,), jnp.float32),
        pltpu.VMEM_SHARED((NG,), jnp.float32),
        pltpu.VMEM((BLK,), jnp.float32),
        pltpu.VMEM((BLK,), jnp.float32),
        pltpu.VMEM((BLK,), jnp.int32),
        pltpu.VMEM((BLK,), jnp.int32),
        pltpu.VMEM((BLK,), jnp.int32),
        pltpu.VMEM((BLK,), jnp.int32),
        pltpu.VMEM((BLK,), jnp.int32),
        pltpu.VMEM((BLK,), jnp.int32),
        pltpu.VMEM((BLK,), jnp.float32),
        pltpu.VMEM((BLK,), jnp.float32),
        pltpu.VMEM((BLK,), jnp.float32),
        pltpu.VMEM((2, KF, 128), jnp.int32),
        pltpu.SemaphoreType.DMA,
        pltpu.SemaphoreType.DMA,
        pltpu.SemaphoreType.DMA,
    ],
)


def _stats_body(p_ref, cen_ref, std_ref, frac_ref):
    p = p_ref[...]
    cnt = p[0] + p[3]
    sx = p[1] + p[4]
    sxx = p[2] + p[5]
    denom = jnp.maximum(cnt, 1.0)
    cen = sx / denom
    var = jnp.maximum(sxx / denom - cen * cen, 0.0)
    std = jnp.sqrt(var)
    nh = jnp.sum(cnt, axis=1, keepdims=True)
    frac = cnt / jnp.maximum(nh, 1.0)
    cen_ref[...] = cen
    std_ref[...] = std
    frac_ref[...] = frac


_stats = pl.pallas_call(
    _stats_body,
    out_shape=(
        jax.ShapeDtypeStruct((N_EVENTS, 80), jnp.float32),
        jax.ShapeDtypeStruct((N_EVENTS, 80), jnp.float32),
        jax.ShapeDtypeStruct((N_EVENTS, 80), jnp.float32),
    ),
)


def _gather_body(xy_hbm, g_hbm, ctab_hbm, ptab_hbm,
                 feat_hbm, dist_hbm,
                 tab, xyb0, xyb1, gb0, gb1, db0, db1, fs0, fs1,
                 insem, outsem):
    cid = lax.axis_index("c")
    sid = lax.axis_index("s")
    wid = sid * NC + cid
    nblk = BASE_BLKS + jnp.where(wid < EXTRA, 1, 0)
    xyb = (xyb0, xyb1)
    gb = (gb0, gb1)
    db = (db0, db1)
    fs = (fs0, fs1)
    himask = jnp.full((16,), -65536, jnp.int32)
    sh16 = jnp.full((16,), 16, jnp.int32)
    iota3 = lax.iota(jnp.int32, 16) * 3
    c1i = jnp.full((16,), 1, jnp.int32)
    c2i = jnp.full((16,), 2, jnp.int32)

    def _hb(i):
        return (wid + NW * i) * BLK

    pltpu.sync_copy(ctab_hbm, tab)

    def _fire1(i, slot):
        sl = pl.ds(_hb(i), BLK)
        pltpu.async_copy(xy_hbm.at[sl], xyb[slot], insem)
        pltpu.async_copy(g_hbm.at[sl], gb[slot], insem)

    def _wait1(slot):
        sl = pl.ds(0, BLK)
        pltpu.make_async_copy(xy_hbm.at[sl], xyb[slot], insem).wait()
        pltpu.make_async_copy(g_hbm.at[sl], gb[slot], insem).wait()

    def _step1(i, slot, fire_next, wait_out):
        @pl.when(wait_out)
        def _():
            pltpu.make_async_copy(
                db[slot], dist_hbm.at[pl.ds(0, BLK)], outsem).wait()

        _wait1(slot)

        @pl.when(fire_next)
        def _():
            _fire1(i + 1, 1 - slot)

        def _vreg(j, _):
            sl16 = pl.ds(j * 16, 16)
            g = gb[slot][sl16]
            cv = plsc.load_gather(tab, [g])
            db[slot][sl16] = jnp.abs(xyb[slot][sl16] - cv)
            return 0

        lax.fori_loop(0, BLK // 16, _vreg, 0)
        pltpu.async_copy(db[slot], dist_hbm.at[pl.ds(_hb(i), BLK)], outsem)

    _fire1(0, 0)

    def _pair1(i2, _):
        i = i2 * 2
        _step1(i, 0, i + 1 < nblk, i >= 2)
        _step1(i + 1, 1, i + 2 < nblk, i >= 1)
        return 0

    lax.fori_loop(0, nblk // 2, _pair1, 0)

    @pl.when(lax.rem(nblk, 2) == 1)
    def _():
        _step1(nblk - 1, 0, jnp.bool_(False), nblk - 1 >= 2)

    pltpu.make_async_copy(db[0], dist_hbm.at[pl.ds(0, BLK)], outsem).wait()
    pltpu.make_async_copy(db[1], dist_hbm.at[pl.ds(0, BLK)], outsem).wait()

    pltpu.sync_copy(ptab_hbm, tab)

    def _fire2(i, slot):
        sl = pl.ds(_hb(i), BLK)
        pltpu.async_copy(g_hbm.at[sl], gb[slot], insem)
        pltpu.async_copy(dist_hbm.at[sl], db[slot], insem)

    def _wait2(slot):
        sl = pl.ds(0, BLK)
        pltpu.make_async_copy(g_hbm.at[sl], gb[slot], insem).wait()
        pltpu.make_async_copy(dist_hbm.at[sl], db[slot], insem).wait()

    def _step2(i, slot, fire_next, wait_out):
        @pl.when(wait_out)
        def _():
            pltpu.make_async_copy(
                fs[slot], feat_hbm.at[pl.ds(0, 3 * BLK)], outsem).wait()

        _wait2(slot)

        @pl.when(fire_next)
        def _():
            _fire2(i + 1, 1 - slot)

        def _vreg(j, _):
            sl16 = pl.ds(j * 16, 16)
            g = gb[slot][sl16]
            w = plsc.bitcast(plsc.load_gather(tab, [g]), jnp.int32)
            s = plsc.bitcast(w & himask, jnp.float32)
            f = plsc.bitcast(lax.shift_left(w, sh16), jnp.float32)
            d = db[slot][sl16]
            h3 = iota3 + j * 48
            plsc.store_scatter(fs[slot], [h3], d)
            plsc.store_scatter(fs[slot], [h3 + c1i], s)
            plsc.store_scatter(fs[slot], [h3 + c2i], f)
            return 0

        lax.fori_loop(0, BLK // 16, _vreg, 0)
        pltpu.async_copy(
            fs[slot], feat_hbm.at[pl.ds(3 * _hb(i), 3 * BLK)], outsem)

    _fire2(0, 0)

    def _pair2(i2, _):
        i = i2 * 2
        _step2(i, 0, i + 1 < nblk, i >= 2)
        _step2(i + 1, 1, i + 2 < nblk, i >= 1)
        return 0

    lax.fori_loop(0, nblk // 2, _pair2, 0)

    @pl.when(lax.rem(nblk, 2) == 1)
    def _():
        _step2(nblk - 1, 0, jnp.bool_(False), nblk - 1 >= 2)

    for slot in range(2):
        pltpu.make_async_copy(
            fs[slot], feat_hbm.at[pl.ds(0, 3 * BLK)], outsem).wait()


_gather = pl.kernel(
    _gather_body,
    out_type=(
        jax.ShapeDtypeStruct((3 * N,), jnp.float32),
        jax.ShapeDtypeStruct((N,), jnp.float32),
    ),
    mesh=_mesh,
    compiler_params=pltpu.CompilerParams(needs_layout_passes=False),
    scratch_types=[
        pltpu.VMEM((NG,), jnp.float32),
        pltpu.VMEM((BLK,), jnp.float32),
        pltpu.VMEM((BLK,), jnp.float32),
        pltpu.VMEM((BLK,), jnp.int32),
        pltpu.VMEM((BLK,), jnp.int32),
        pltpu.VMEM((BLK,), jnp.float32),
        pltpu.VMEM((BLK,), jnp.float32),
        pltpu.VMEM((3 * BLK,), jnp.float32),
        pltpu.VMEM((3 * BLK,), jnp.float32),
        pltpu.SemaphoreType.DMA,
        pltpu.SemaphoreType.DMA,
    ],
)


@jax.jit
def kernel(xy, z_q, ori, batch_idx):
    z = z_q.astype(jnp.int32)
    o = ori.astype(jnp.int32)
    b = batch_idx.astype(jnp.int32)
    zero = jnp.zeros((NG // NS,), jnp.float32)

    partials, gids = _accum(xy, z, o, b, zero)
    p = partials.reshape(6, N_EVENTS, 80)
    cen, std, frac = _stats(p)

    su = lax.bitcast_convert_type(std.reshape(NG), jnp.uint32)
    fu = lax.bitcast_convert_type(frac.reshape(NG), jnp.uint32)
    packed = ((su + 0x8000) & jnp.uint32(0xFFFF0000)) | ((fu + 0x8000) >> 16)
    ptab = lax.bitcast_convert_type(packed, jnp.float32)

    feat, _unused_dist = _gather(xy, gids.reshape(N), cen.reshape(NG), ptab)
    return feat.reshape(N, 3)


# speedup vs baseline: 9.0874x; 9.0874x over previous
import jax
import jax.numpy as jnp
from jax import lax
from jax.experimental import pallas as pl
from jax.experimental.pallas import tpu as pltpu
from jax.experimental.pallas import tpu_sc as plsc

N = 6400000
N_EVENTS = 1024
N_STATIONS = 40
N_ORI = 2
NG = N_EVENTS * N_STATIONS * N_ORI

NC = 2
NS = 16
NW = NC * NS

BLK = 2048
NBLK = N // BLK
BASE_BLKS = NBLK // NW
EXTRA = NBLK - BASE_BLKS * NW
KF = BLK // 128

_mesh = plsc.VectorSubcoreMesh(core_axis_name="c", subcore_axis_name="s")


def _accum_body(xy_hbm, z_hbm, o_hbm, b_hbm, zero_hbm, out_hbm, g_hbm,
                cnt_t, sum_t, sq_t, xyb0, xyb1, zb0, zb1, ob0, ob1, bb0, bb1,
                sqb0, sqb1, onesb, idx128, insem, ssem, gsem):
    cid = lax.axis_index("c")
    sid = lax.axis_index("s")
    wid = sid * NC + cid
    nblk = BASE_BLKS + jnp.where(wid < EXTRA, 1, 0)
    xyb = (xyb0, xyb1)
    zb = (zb0, zb1)
    ob = (ob0, ob1)
    bb = (bb0, bb1)
    sqb = (sqb0, sqb1)
    ones = jnp.full((16,), 1.0, jnp.float32)

    zrows = NG // NS
    for t in (cnt_t, sum_t, sq_t):
        pltpu.sync_copy(zero_hbm, t.at[pl.ds(sid * zrows, zrows)])

    def _prefill(j, _):
        onesb[pl.ds(j * 16, 16)] = ones
        return 0

    lax.fori_loop(0, BLK // 16, _prefill, 0)

    plsc.subcore_barrier()

    def _hb(i):
        return (wid + NW * i) * BLK

    def _fire_in(i, slot):
        sl = pl.ds(_hb(i), BLK)
        pltpu.async_copy(xy_hbm.at[sl], xyb[slot], insem)
        pltpu.async_copy(z_hbm.at[sl], zb[slot], insem)
        pltpu.async_copy(o_hbm.at[sl], ob[slot], insem)
        pltpu.async_copy(b_hbm.at[sl], bb[slot], insem)

    def _wait_in(slot):
        sl = pl.ds(0, BLK)
        pltpu.make_async_copy(xy_hbm.at[sl], xyb[slot], insem).wait()
        pltpu.make_async_copy(z_hbm.at[sl], zb[slot], insem).wait()
        pltpu.make_async_copy(o_hbm.at[sl], ob[slot], insem).wait()
        pltpu.make_async_copy(b_hbm.at[sl], bb[slot], insem).wait()

    def _step(i, slot, fire_next):
        _wait_in(slot)

        @pl.when(fire_next)
        def _():
            _fire_in(i + 1, 1 - slot)

        def _vreg(j, _):
            sl16 = pl.ds(j * 16, 16)
            b = bb[slot][sl16]
            z = zb[slot][sl16]
            o = ob[slot][sl16]
            xv = xyb[slot][sl16]
            g = (b * N_STATIONS + z) * N_ORI + o
            idx128[slot, j // 8, pl.ds((j % 8) * 16, 16)] = g
            sqb[slot][sl16] = xv * xv
            return 0

        descs = []
        for m in range(4):
            lax.fori_loop(m * (BLK // 64), (m + 1) * (BLK // 64), _vreg, 0)
            for k in range(m * (KF // 4), (m + 1) * (KF // 4)):
                sl = pl.ds(k * 128, 128)
                irow = idx128.at[slot, k]
                descs.append(pltpu.async_copy(
                    onesb.at[sl], cnt_t.at[irow], ssem, add=True))
                descs.append(pltpu.async_copy(
                    xyb[slot].at[sl], sum_t.at[irow], ssem, add=True))
                descs.append(pltpu.async_copy(
                    sqb[slot].at[sl], sq_t.at[irow], ssem, add=True))

        gdst = g_hbm.at[pl.ds((wid + NW * i) * KF, KF), :]
        pltpu.async_copy(idx128.at[slot, pl.ds(0, KF)], gdst, gsem)

        for d in descs:
            d.wait()
        pltpu.make_async_copy(
            idx128.at[slot, pl.ds(0, KF)], gdst, gsem).wait()

    _fire_in(0, 0)

    def _pair(i2, _):
        i = i2 * 2
        _step(i, 0, i + 1 < nblk)
        _step(i + 1, 1, i + 2 < nblk)
        return 0

    lax.fori_loop(0, nblk // 2, _pair, 0)

    @pl.when(lax.rem(nblk, 2) == 1)
    def _():
        _step(nblk - 1, 0, jnp.bool_(False))

    plsc.subcore_barrier()

    for comp, t in enumerate((cnt_t, sum_t, sq_t)):
        pltpu.sync_copy(
            t.at[pl.ds(sid * zrows, zrows)],
            out_hbm.at[pl.ds((cid * 3 + comp) * NG + sid * zrows, zrows)])


_accum = pl.kernel(
    _accum_body,
    out_type=(jax.ShapeDtypeStruct((NC * 3 * NG,), jnp.float32),
              jax.ShapeDtypeStruct((N // 128, 128), jnp.int32)),
    mesh=_mesh,
    compiler_params=pltpu.CompilerParams(needs_layout_passes=False),
    scratch_types=[
        pltpu.VMEM_SHARED((NG,), jnp.float32),
        pltpu.VMEM_SHARED((NG,), jnp.float32),
        pltpu.VMEM_SHARED((NG,), jnp.float32),
        pltpu.VMEM((BLK,), jnp.float32),
        pltpu.VMEM((BLK,), jnp.float32),
        pltpu.VMEM((BLK,), jnp.int32),
        pltpu.VMEM((BLK,), jnp.int32),
        pltpu.VMEM((BLK,), jnp.int32),
        pltpu.VMEM((BLK,), jnp.int32),
        pltpu.VMEM((BLK,), jnp.int32),
        pltpu.VMEM((BLK,), jnp.int32),
        pltpu.VMEM((BLK,), jnp.float32),
        pltpu.VMEM((BLK,), jnp.float32),
        pltpu.VMEM((BLK,), jnp.float32),
        pltpu.VMEM((2, KF, 128), jnp.int32),
        pltpu.SemaphoreType.DMA,
        pltpu.SemaphoreType.DMA,
        pltpu.SemaphoreType.DMA,
    ],
)


def _stats_body(p_ref, cen_ref, std_ref, frac_ref):
    p = p_ref[...]
    cnt = p[0] + p[3]
    sx = p[1] + p[4]
    sxx = p[2] + p[5]
    denom = jnp.maximum(cnt, 1.0)
    cen = sx / denom
    var = jnp.maximum(sxx / denom - cen * cen, 0.0)
    std = jnp.sqrt(var)
    nh = jnp.sum(cnt, axis=1, keepdims=True)
    frac = cnt / jnp.maximum(nh, 1.0)
    cen_ref[...] = cen
    std_ref[...] = std
    frac_ref[...] = frac


_stats = pl.pallas_call(
    _stats_body,
    out_shape=(
        jax.ShapeDtypeStruct((N_EVENTS, 80), jnp.float32),
        jax.ShapeDtypeStruct((N_EVENTS, 80), jnp.float32),
        jax.ShapeDtypeStruct((N_EVENTS, 80), jnp.float32),
    ),
)


def _gather_body(xy_hbm, g_hbm, ctab_hbm, ptab_hbm,
                 dist_hbm, std_hbm, frac_hbm,
                 tab, xyb0, xyb1, gb0, gb1, db0, db1, sb0, sb1, fb0, fb1,
                 insem, outsem):
    cid = lax.axis_index("c")
    sid = lax.axis_index("s")
    wid = sid * NC + cid
    nblk = BASE_BLKS + jnp.where(wid < EXTRA, 1, 0)
    xyb = (xyb0, xyb1)
    gb = (gb0, gb1)
    db = (db0, db1)
    sb = (sb0, sb1)
    fb = (fb0, fb1)
    himask = jnp.full((16,), -65536, jnp.int32)
    sh16 = jnp.full((16,), 16, jnp.int32)

    def _hb(i):
        return (wid + NW * i) * BLK

    pltpu.sync_copy(ctab_hbm, tab)

    def _fire1(i, slot):
        sl = pl.ds(_hb(i), BLK)
        pltpu.async_copy(xy_hbm.at[sl], xyb[slot], insem)
        pltpu.async_copy(g_hbm.at[sl], gb[slot], insem)

    def _wait1(slot):
        sl = pl.ds(0, BLK)
        pltpu.make_async_copy(xy_hbm.at[sl], xyb[slot], insem).wait()
        pltpu.make_async_copy(g_hbm.at[sl], gb[slot], insem).wait()

    def _step1(i, slot, fire_next, wait_out):
        @pl.when(wait_out)
        def _():
            pltpu.make_async_copy(
                db[slot], dist_hbm.at[pl.ds(0, BLK)], outsem).wait()

        _wait1(slot)

        @pl.when(fire_next)
        def _():
            _fire1(i + 1, 1 - slot)

        def _vreg(j, _):
            sl16 = pl.ds(j * 16, 16)
            g = gb[slot][sl16]
            cv = plsc.load_gather(tab, [g])
            db[slot][sl16] = jnp.abs(xyb[slot][sl16] - cv)
            return 0

        lax.fori_loop(0, BLK // 16, _vreg, 0)
        pltpu.async_copy(db[slot], dist_hbm.at[pl.ds(_hb(i), BLK)], outsem)

    _fire1(0, 0)

    def _pair1(i2, _):
        i = i2 * 2
        _step1(i, 0, i + 1 < nblk, i >= 2)
        _step1(i + 1, 1, i + 2 < nblk, i >= 1)
        return 0

    lax.fori_loop(0, nblk // 2, _pair1, 0)

    @pl.when(lax.rem(nblk, 2) == 1)
    def _():
        _step1(nblk - 1, 0, jnp.bool_(False), nblk - 1 >= 2)

    pltpu.make_async_copy(db[0], dist_hbm.at[pl.ds(0, BLK)], outsem).wait()
    pltpu.make_async_copy(db[1], dist_hbm.at[pl.ds(0, BLK)], outsem).wait()

    pltpu.sync_copy(ptab_hbm, tab)

    def _fire2(i, slot):
        pltpu.async_copy(g_hbm.at[pl.ds(_hb(i), BLK)], gb[slot], insem)

    def _wait2(slot):
        pltpu.make_async_copy(
            g_hbm.at[pl.ds(0, BLK)], gb[slot], insem).wait()

    def _step2(i, slot, fire_next, wait_out):
        @pl.when(wait_out)
        def _():
            pltpu.make_async_copy(
                sb[slot], std_hbm.at[pl.ds(0, BLK)], outsem).wait()
            pltpu.make_async_copy(
                fb[slot], frac_hbm.at[pl.ds(0, BLK)], outsem).wait()

        _wait2(slot)

        @pl.when(fire_next)
        def _():
            _fire2(i + 1, 1 - slot)

        def _vreg(j, _):
            sl16 = pl.ds(j * 16, 16)
            g = gb[slot][sl16]
            w = plsc.bitcast(plsc.load_gather(tab, [g]), jnp.int32)
            sb[slot][sl16] = plsc.bitcast(w & himask, jnp.float32)
            fb[slot][sl16] = plsc.bitcast(
                lax.shift_left(w, sh16), jnp.float32)
            return 0

        lax.fori_loop(0, BLK // 16, _vreg, 0)
        sl = pl.ds(_hb(i), BLK)
        pltpu.async_copy(sb[slot], std_hbm.at[sl], outsem)
        pltpu.async_copy(fb[slot], frac_hbm.at[sl], outsem)

    _fire2(0, 0)

    def _pair2(i2, _):
        i = i2 * 2
        _step2(i, 0, i + 1 < nblk, i >= 2)
        _step2(i + 1, 1, i + 2 < nblk, i >= 1)
        return 0

    lax.fori_loop(0, nblk // 2, _pair2, 0)

    @pl.when(lax.rem(nblk, 2) == 1)
    def _():
        _step2(nblk - 1, 0, jnp.bool_(False), nblk - 1 >= 2)

    for slot in range(2):
        pltpu.make_async_copy(
            sb[slot], std_hbm.at[pl.ds(0, BLK)], outsem).wait()
        pltpu.make_async_copy(
            fb[slot], frac_hbm.at[pl.ds(0, BLK)], outsem).wait()


_gather = pl.kernel(
    _gather_body,
    out_type=(
        jax.ShapeDtypeStruct((N,), jnp.float32),
        jax.ShapeDtypeStruct((N,), jnp.float32),
        jax.ShapeDtypeStruct((N,), jnp.float32),
    ),
    mesh=_mesh,
    compiler_params=pltpu.CompilerParams(needs_layout_passes=False),
    scratch_types=[
        pltpu.VMEM((NG,), jnp.float32),
        pltpu.VMEM((BLK,), jnp.float32),
        pltpu.VMEM((BLK,), jnp.float32),
        pltpu.VMEM((BLK,), jnp.int32),
        pltpu.VMEM((BLK,), jnp.int32),
        pltpu.VMEM((BLK,), jnp.float32),
        pltpu.VMEM((BLK,), jnp.float32),
        pltpu.VMEM((BLK,), jnp.float32),
        pltpu.VMEM((BLK,), jnp.float32),
        pltpu.VMEM((BLK,), jnp.float32),
        pltpu.VMEM((BLK,), jnp.float32),
        pltpu.SemaphoreType.DMA,
        pltpu.SemaphoreType.DMA,
    ],
)


@jax.jit
def kernel(xy, z_q, ori, batch_idx):
    z = z_q.astype(jnp.int32)
    o = ori.astype(jnp.int32)
    b = batch_idx.astype(jnp.int32)
    zero = jnp.zeros((NG // NS,), jnp.float32)

    partials, gids = _accum(xy, z, o, b, zero)
    p = partials.reshape(6, N_EVENTS, 80)
    cen, std, frac = _stats(p)

    su = lax.bitcast_convert_type(std.reshape(NG), jnp.uint32)
    fu = lax.bitcast_convert_type(frac.reshape(NG), jnp.uint32)
    packed = ((su + 0x8000) & jnp.uint32(0xFFFF0000)) | ((fu + 0x8000) >> 16)
    ptab = lax.bitcast_convert_type(packed, jnp.float32)

    dist, stdo, fraco = _gather(xy, gids.reshape(N), cen.reshape(NG), ptab)
    return jnp.stack([dist, stdo, fraco], axis=-1)
